# Initial kernel scaffold; baseline (speedup 1.0000x reference)
import jax
import jax.numpy as jnp
from jax.experimental import pallas as pl


def kernel(x, edge_index, edge_weight, edge_attr, W0, b0, Wc0, Wc1):
    raise NotImplementedError("write your pallas kernel here")


# speedup vs baseline: 6.7769x; 6.7769x over previous
import functools

import jax
import jax.numpy as jnp
from jax import lax
from jax.experimental import pallas as pl
from jax.experimental.pallas import tpu as pltpu
from jax.experimental.pallas import tpu_sc as plsc

N_NODES = 10000
N_PAD = 10240
C_FEAT = 128
N_EDGES = 320000
ALPHA_C = 0.9

_INFO = plsc.get_sparse_core_info()
NC = _INFO.num_cores
NS = _INFO.num_subcores
NW = NC * NS
EPW = N_EDGES // NW
EPW_P = 10240
CHUNK = 80
NCHUNK = EPW_P // CHUNK
GRP = 8
NGRP = NCHUNK // GRP
RPT = N_PAD // NS
BLK = 1024


@functools.partial(
    pl.kernel,
    out_type=jax.ShapeDtypeStruct((NW, 8, N_PAD), jnp.float32),
    mesh=plsc.VectorSubcoreMesh(core_axis_name="c", subcore_axis_name="s"),
    compiler_params=pltpu.CompilerParams(needs_layout_passes=False),
    scratch_types=[
        pltpu.VMEM((8, EPW_P // 8), jnp.int32),
        pltpu.VMEM((8, EPW_P // 8), jnp.float32),
        pltpu.VMEM((1, N_PAD), jnp.float32),
    ],
)
def _deg_kernel(col_hbm, ew_hbm, out_hbm, col_v, ew_v, deg_v):
    cid = lax.axis_index("c")
    sid = lax.axis_index("s")
    wid = cid * NS + sid

    zeros16f = jnp.zeros((16,), jnp.float32)
    zeros16i = jnp.zeros((16,), jnp.int32)

    def zero_body(i, carry):
        deg_v[0, pl.ds(pl.multiple_of(i * 16, 16), 16)] = zeros16f
        return carry

    lax.fori_loop(0, N_PAD // 16, zero_body, 0)

    pltpu.sync_copy(col_hbm.at[wid], col_v)
    pltpu.sync_copy(ew_hbm.at[wid], ew_v)

    for r in range(8):
        def acc_body(i, carry):
            off = pl.multiple_of(i * 16, 16)
            idx = col_v[r, pl.ds(off, 16)]
            w = ew_v[r, pl.ds(off, 16)]
            plsc.addupdate_scatter(deg_v, [zeros16i, idx], w)
            return carry

        lax.fori_loop(0, EPW_P // 8 // 16, acc_body, 0)

    pltpu.sync_copy(deg_v, out_hbm.at[wid, pl.ds(0, 1)])


@functools.partial(
    pl.kernel,
    out_type=jax.ShapeDtypeStruct((NC, N_PAD, C_FEAT), jnp.float32),
    mesh=plsc.VectorSubcoreMesh(core_axis_name="c", subcore_axis_name="s"),
    compiler_params=pltpu.CompilerParams(needs_layout_passes=False),
    scratch_types=[
        pltpu.VMEM((GRP, CHUNK), jnp.int32),
        pltpu.VMEM((GRP, CHUNK), jnp.int32),
        pltpu.VMEM((GRP, CHUNK), jnp.float32),
        pltpu.VMEM((N_PAD,), jnp.float32),
        pltpu.VMEM((CHUNK,), jnp.float32),
        pltpu.VMEM((CHUNK, C_FEAT), jnp.float32),
        pltpu.VMEM_SHARED((N_PAD, C_FEAT), jnp.float32),
        pltpu.SemaphoreType.DMA,
    ],
)
def _spmm_kernel(h_hbm, dis_hbm, row_hbm, col_hbm, ew_hbm, out_hbm,
                 row_v, col_v, ew_v, dis_v, nrm_v, gbuf, accum, sem):
    cid = lax.axis_index("c")
    sid = lax.axis_index("s")
    wid = cid * NS + sid

    pltpu.sync_copy(dis_hbm, dis_v)

    zeros16 = jnp.zeros((16,), jnp.float32)

    def zzero(i, carry):
        for q in range(C_FEAT // 16):
            gbuf[i, pl.ds(q * 16, 16)] = zeros16
        return carry

    lax.fori_loop(0, CHUNK, zzero, 0)
    for k in range(RPT // CHUNK):
        r0 = pl.multiple_of(sid * RPT + k * CHUNK, 8)
        pltpu.sync_copy(gbuf, accum.at[pl.ds(r0, CHUNK)])
    plsc.subcore_barrier()

    def grp_body(g, carry):
        g0 = pl.multiple_of(g * GRP, 8)
        pltpu.sync_copy(row_hbm.at[wid, pl.ds(g0, GRP)], row_v)
        pltpu.sync_copy(col_hbm.at[wid, pl.ds(g0, GRP)], col_v)
        pltpu.sync_copy(ew_hbm.at[wid, pl.ds(g0, GRP)], ew_v)

        def chunk_body(j, c1):
            pltpu.async_copy(h_hbm.at[row_v.at[j]], gbuf, sem).wait()

            for i in range(CHUNK // 16):
                idr = row_v[j, pl.ds(i * 16, 16)]
                idc = col_v[j, pl.ds(i * 16, 16)]
                w16 = ew_v[j, pl.ds(i * 16, 16)]
                dr = plsc.load_gather(dis_v, [idr])
                dc = plsc.load_gather(dis_v, [idc])
                nrm_v[pl.ds(i * 16, 16)] = dr * w16 * dc

            def scale_body(i, c2):
                off = pl.multiple_of(i * 16, 16)
                nv16 = nrm_v[pl.ds(off, 16)]
                for l in range(16):
                    nv = jnp.full((16,), nv16[l], jnp.float32)
                    e = off + l
                    for q in range(C_FEAT // 16):
                        gbuf[e, pl.ds(q * 16, 16)] = (
                            gbuf[e, pl.ds(q * 16, 16)] * nv)
                return c2

            lax.fori_loop(0, CHUNK // 16, scale_body, 0)

            pltpu.sync_copy(gbuf, accum.at[col_v.at[j]], add=True)
            return c1

        lax.fori_loop(0, GRP, chunk_body, 0)
        return carry

    lax.fori_loop(0, NGRP, grp_body, 0)
    plsc.subcore_barrier()

    def wout(k, carry):
        r0 = pl.multiple_of(sid * RPT + k * CHUNK, 8)
        pltpu.sync_copy(accum.at[pl.ds(r0, CHUNK)],
                        out_hbm.at[cid, pl.ds(r0, CHUNK)])
        return carry

    lax.fori_loop(0, RPT // CHUNK, wout, 0)


def _h_body(x_ref, w_ref, b_ref, o_ref):
    o_ref[...] = jnp.maximum(
        jnp.dot(x_ref[...], w_ref[...], preferred_element_type=jnp.float32)
        + b_ref[...], 0.0)


def _h_kernel(x, W0, b0):
    return pl.pallas_call(
        _h_body,
        grid=(N_PAD // BLK,),
        in_specs=[
            pl.BlockSpec((BLK, C_FEAT), lambda i: (i, 0)),
            pl.BlockSpec((C_FEAT, C_FEAT), lambda i: (0, 0)),
            pl.BlockSpec((1, C_FEAT), lambda i: (0, 0)),
        ],
        out_specs=pl.BlockSpec((BLK, C_FEAT), lambda i: (i, 0)),
        out_shape=jax.ShapeDtypeStruct((N_PAD, C_FEAT), jnp.float32),
    )(x, W0, b0)


def _dis_body(degp_ref, o_ref):
    deg = jnp.sum(degp_ref[:, 0, :], axis=0, keepdims=True)
    safe = jnp.where(deg > 0, deg, 1.0)
    o_ref[...] = jnp.where(deg > 0, lax.rsqrt(safe), 0.0)


def _dis_kernel(degp):
    return pl.pallas_call(
        _dis_body,
        grid=(1,),
        in_specs=[pl.BlockSpec((NW, 8, N_PAD), lambda i: (0, 0, 0))],
        out_specs=pl.BlockSpec((1, N_PAD), lambda i: (0, 0)),
        out_shape=jax.ShapeDtypeStruct((1, N_PAD), jnp.float32),
    )(degp)


def _upd_body(p_ref, h_ref, prev_ref, w_ref, o_ref):
    agg = p_ref[0, :, :] + p_ref[1, :, :]
    t = (1.0 - ALPHA_C) * agg + ALPHA_C * h_ref[...]
    o_ref[...] = prev_ref[...] + jnp.maximum(
        jnp.dot(t, w_ref[...], preferred_element_type=jnp.float32), 0.0)


def _upd_kernel(p, h, prev, W):
    return pl.pallas_call(
        _upd_body,
        grid=(N_PAD // BLK,),
        in_specs=[
            pl.BlockSpec((NC, BLK, C_FEAT), lambda i: (0, i, 0)),
            pl.BlockSpec((BLK, C_FEAT), lambda i: (i, 0)),
            pl.BlockSpec((BLK, C_FEAT), lambda i: (i, 0)),
            pl.BlockSpec((C_FEAT, C_FEAT), lambda i: (0, 0)),
        ],
        out_specs=pl.BlockSpec((BLK, C_FEAT), lambda i: (i, 0)),
        out_shape=jax.ShapeDtypeStruct((N_PAD, C_FEAT), jnp.float32),
    )(p, h, prev, W)


def kernel(x, edge_index, edge_weight, edge_attr, W0, b0, Wc0, Wc1):
    row2 = edge_index[0].reshape(NW, EPW)
    col2 = edge_index[1].reshape(NW, EPW)
    ew2 = edge_weight.reshape(NW, EPW)
    pad = ((0, 0), (0, EPW_P - EPW))
    rowp = jnp.pad(row2, pad)
    colp = jnp.pad(col2, pad)
    ewp = jnp.pad(ew2, pad)

    row3 = rowp.reshape(NW, NCHUNK, CHUNK)
    col3 = colp.reshape(NW, NCHUNK, CHUNK)
    ew3 = ewp.reshape(NW, NCHUNK, CHUNK)
    col4 = colp.reshape(NW, 8, EPW_P // 8)
    ew4 = ewp.reshape(NW, 8, EPW_P // 8)
    x_pad = jnp.pad(x, ((0, N_PAD - N_NODES), (0, 0)))

    h = _h_kernel(x_pad, W0, b0.reshape(1, C_FEAT))
    degp = _deg_kernel(col4, ew4)
    dis = _dis_kernel(degp).reshape(N_PAD)

    p1 = _spmm_kernel(h, dis, row3, col3, ew3)
    out1 = _upd_kernel(p1, h, h, Wc0)
    p2 = _spmm_kernel(out1, dis, row3, col3, ew3)
    out2 = _upd_kernel(p2, h, out1, Wc1)
    return out2[:N_NODES]


# speedup vs baseline: 8.4987x; 1.2541x over previous
import functools

import jax
import jax.numpy as jnp
from jax import lax
from jax.experimental import pallas as pl
from jax.experimental.pallas import tpu as pltpu
from jax.experimental.pallas import tpu_sc as plsc

N_NODES = 10000
N_PAD = 10240
C_FEAT = 128
N_EDGES = 320000
ALPHA_C = 0.9

_INFO = plsc.get_sparse_core_info()
NC = _INFO.num_cores
NS = _INFO.num_subcores
NW = NC * NS
EPW = N_EDGES // NW
EPW_P = 10240
CHUNK = 80
NCHUNK = EPW_P // CHUNK
GRP = 8
NGRP = NCHUNK // GRP
RPT = N_PAD // NS
BLK = 1024


@functools.partial(
    pl.kernel,
    out_type=jax.ShapeDtypeStruct((NW, 8, N_PAD), jnp.float32),
    mesh=plsc.VectorSubcoreMesh(core_axis_name="c", subcore_axis_name="s"),
    compiler_params=pltpu.CompilerParams(needs_layout_passes=False),
    scratch_types=[
        pltpu.VMEM((8, EPW_P // 8), jnp.int32),
        pltpu.VMEM((8, EPW_P // 8), jnp.float32),
        pltpu.VMEM((1, N_PAD), jnp.float32),
    ],
)
def _deg_kernel(col_hbm, ew_hbm, out_hbm, col_v, ew_v, deg_v):
    cid = lax.axis_index("c")
    sid = lax.axis_index("s")
    wid = cid * NS + sid

    zeros16f = jnp.zeros((16,), jnp.float32)
    zeros16i = jnp.zeros((16,), jnp.int32)

    def zero_body(i, carry):
        deg_v[0, pl.ds(pl.multiple_of(i * 16, 16), 16)] = zeros16f
        return carry

    lax.fori_loop(0, N_PAD // 16, zero_body, 0)

    pltpu.sync_copy(col_hbm.at[wid], col_v)
    pltpu.sync_copy(ew_hbm.at[wid], ew_v)

    for r in range(8):
        def acc_body(i, carry):
            off = pl.multiple_of(i * 16, 16)
            idx = col_v[r, pl.ds(off, 16)]
            w = ew_v[r, pl.ds(off, 16)]
            plsc.addupdate_scatter(deg_v, [zeros16i, idx], w)
            return carry

        lax.fori_loop(0, EPW_P // 8 // 16, acc_body, 0)

    pltpu.sync_copy(deg_v, out_hbm.at[wid, pl.ds(0, 1)])


@functools.partial(
    pl.kernel,
    out_type=jax.ShapeDtypeStruct((NC, N_PAD, C_FEAT), jnp.float32),
    mesh=plsc.VectorSubcoreMesh(core_axis_name="c", subcore_axis_name="s"),
    compiler_params=pltpu.CompilerParams(needs_layout_passes=False),
    scratch_types=[
        pltpu.VMEM((2, GRP, CHUNK), jnp.int32),
        pltpu.VMEM((2, GRP, CHUNK), jnp.int32),
        pltpu.VMEM((2, GRP, CHUNK), jnp.float32),
        pltpu.VMEM((N_PAD,), jnp.float32),
        pltpu.VMEM((CHUNK,), jnp.float32),
        pltpu.VMEM((CHUNK, C_FEAT), jnp.float32),
        pltpu.VMEM((CHUNK, C_FEAT), jnp.float32),
        pltpu.VMEM_SHARED((N_PAD, C_FEAT), jnp.float32),
        pltpu.SemaphoreType.DMA,
        pltpu.SemaphoreType.DMA,
        pltpu.SemaphoreType.DMA,
        pltpu.SemaphoreType.DMA,
        pltpu.SemaphoreType.DMA,
    ],
)
def _spmm_kernel(h_hbm, dis_hbm, row_hbm, col_hbm, ew_hbm, out_hbm,
                 rv3, cv3, wv3, dis_v, nrm_v, gb0, gb1, accum,
                 gsem0, gsem1, ssem0, ssem1, stsem):
    cid = lax.axis_index("c")
    sid = lax.axis_index("s")
    wid = cid * NS + sid
    gb = (gb0, gb1)
    gsem = (gsem0, gsem1)
    ssem = (ssem0, ssem1)

    pltpu.sync_copy(dis_hbm, dis_v)

    zeros16 = jnp.zeros((16,), jnp.float32)

    def zzero(i, carry):
        for q in range(C_FEAT // 16):
            gb0[i, pl.ds(q * 16, 16)] = zeros16
        return carry

    lax.fori_loop(0, CHUNK, zzero, 0)
    for k in range(RPT // CHUNK):
        r0 = pl.multiple_of(sid * RPT + k * CHUNK, 8)
        pltpu.sync_copy(gb0, accum.at[pl.ds(r0, CHUNK)])
    plsc.subcore_barrier()

    def stage_grp(g1, dst, sync):
        o = pl.multiple_of(g1 * GRP, 8)
        copies = ((row_hbm, rv3), (col_hbm, cv3), (ew_hbm, wv3))
        for src, d3 in copies:
            c = pltpu.make_async_copy(src.at[wid, pl.ds(o, GRP)],
                                      d3.at[dst], stsem)
            if sync == "start":
                c.start()
            else:
                c.wait()

    stage_grp(0, 0, "start")
    stage_grp(0, 0, "wait")
    pltpu.async_copy(h_hbm.at[rv3.at[0, 0]], gb0, gsem0)

    def grp_body(g, carry):
        p = g % 2
        np_ = 1 - p

        for jj in range(GRP):
            b = jj % 2
            nb = 1 - b
            pltpu.make_async_copy(
                h_hbm.at[rv3.at[p, jj]], gb[b], gsem[b]).wait()

            if jj == 2:
                pl.when(g < NGRP - 1)(
                    lambda: stage_grp(g + 1, np_, "start"))

            def drain_nb():
                pltpu.make_async_copy(
                    gb[nb], accum.at[cv3.at[p, jj]], ssem[nb]).wait()

            if jj < GRP - 1:
                if jj == 0:
                    pl.when(g > 0)(drain_nb)
                else:
                    drain_nb()
                pltpu.async_copy(h_hbm.at[rv3.at[p, jj + 1]],
                                 gb[nb], gsem[nb])
            else:
                def next_group_gather():
                    stage_grp(g + 1, np_, "wait")
                    drain_nb()
                    pltpu.async_copy(h_hbm.at[rv3.at[np_, 0]],
                                     gb[nb], gsem[nb])
                pl.when(g < NGRP - 1)(next_group_gather)

            for i in range(CHUNK // 16):
                idr = rv3[p, jj, pl.ds(i * 16, 16)]
                idc = cv3[p, jj, pl.ds(i * 16, 16)]
                w16 = wv3[p, jj, pl.ds(i * 16, 16)]
                dr = plsc.load_gather(dis_v, [idr])
                dc = plsc.load_gather(dis_v, [idc])
                nrm_v[pl.ds(i * 16, 16)] = dr * w16 * dc

            gbb = gb[b]

            def scale_body(i, c2):
                off = pl.multiple_of(i * 16, 16)
                nv16 = nrm_v[pl.ds(off, 16)]
                for l in range(16):
                    nv = jnp.full((16,), nv16[l], jnp.float32)
                    e = off + l
                    for q in range(C_FEAT // 16):
                        gbb[e, pl.ds(q * 16, 16)] = (
                            gbb[e, pl.ds(q * 16, 16)] * nv)
                return c2

            lax.fori_loop(0, CHUNK // 16, scale_body, 0)

            pltpu.async_copy(gb[b], accum.at[cv3.at[p, jj]],
                             ssem[b], add=True)
        return carry

    lax.fori_loop(0, NGRP, grp_body, 0)

    pltpu.make_async_copy(gb0, accum.at[cv3.at[0, 0]], ssem0).wait()
    pltpu.make_async_copy(gb1, accum.at[cv3.at[0, 0]], ssem1).wait()
    plsc.subcore_barrier()

    def wout(k, carry):
        r0 = pl.multiple_of(sid * RPT + k * CHUNK, 8)
        pltpu.sync_copy(accum.at[pl.ds(r0, CHUNK)],
                        out_hbm.at[cid, pl.ds(r0, CHUNK)])
        return carry

    lax.fori_loop(0, RPT // CHUNK, wout, 0)


def _h_body(x_ref, w_ref, b_ref, o_ref):
    o_ref[...] = jnp.maximum(
        jnp.dot(x_ref[...], w_ref[...], preferred_element_type=jnp.float32)
        + b_ref[...], 0.0)


def _h_kernel(x, W0, b0):
    return pl.pallas_call(
        _h_body,
        grid=(N_PAD // BLK,),
        in_specs=[
            pl.BlockSpec((BLK, C_FEAT), lambda i: (i, 0)),
            pl.BlockSpec((C_FEAT, C_FEAT), lambda i: (0, 0)),
            pl.BlockSpec((1, C_FEAT), lambda i: (0, 0)),
        ],
        out_specs=pl.BlockSpec((BLK, C_FEAT), lambda i: (i, 0)),
        out_shape=jax.ShapeDtypeStruct((N_PAD, C_FEAT), jnp.float32),
    )(x, W0, b0)


def _dis_body(degp_ref, o_ref):
    deg = jnp.sum(degp_ref[:, 0, :], axis=0, keepdims=True)
    safe = jnp.where(deg > 0, deg, 1.0)
    o_ref[...] = jnp.where(deg > 0, lax.rsqrt(safe), 0.0)


def _dis_kernel(degp):
    return pl.pallas_call(
        _dis_body,
        grid=(1,),
        in_specs=[pl.BlockSpec((NW, 8, N_PAD), lambda i: (0, 0, 0))],
        out_specs=pl.BlockSpec((1, N_PAD), lambda i: (0, 0)),
        out_shape=jax.ShapeDtypeStruct((1, N_PAD), jnp.float32),
    )(degp)


def _upd_body(p_ref, h_ref, prev_ref, w_ref, o_ref):
    agg = p_ref[0, :, :] + p_ref[1, :, :]
    t = (1.0 - ALPHA_C) * agg + ALPHA_C * h_ref[...]
    o_ref[...] = prev_ref[...] + jnp.maximum(
        jnp.dot(t, w_ref[...], preferred_element_type=jnp.float32), 0.0)


def _upd_kernel(p, h, prev, W):
    return pl.pallas_call(
        _upd_body,
        grid=(N_PAD // BLK,),
        in_specs=[
            pl.BlockSpec((NC, BLK, C_FEAT), lambda i: (0, i, 0)),
            pl.BlockSpec((BLK, C_FEAT), lambda i: (i, 0)),
            pl.BlockSpec((BLK, C_FEAT), lambda i: (i, 0)),
            pl.BlockSpec((C_FEAT, C_FEAT), lambda i: (0, 0)),
        ],
        out_specs=pl.BlockSpec((BLK, C_FEAT), lambda i: (i, 0)),
        out_shape=jax.ShapeDtypeStruct((N_PAD, C_FEAT), jnp.float32),
    )(p, h, prev, W)


def kernel(x, edge_index, edge_weight, edge_attr, W0, b0, Wc0, Wc1):
    row2 = edge_index[0].reshape(NW, EPW)
    col2 = edge_index[1].reshape(NW, EPW)
    ew2 = edge_weight.reshape(NW, EPW)
    pad = ((0, 0), (0, EPW_P - EPW))
    rowp = jnp.pad(row2, pad)
    colp = jnp.pad(col2, pad)
    ewp = jnp.pad(ew2, pad)

    row3 = rowp.reshape(NW, NCHUNK, CHUNK)
    col3 = colp.reshape(NW, NCHUNK, CHUNK)
    ew3 = ewp.reshape(NW, NCHUNK, CHUNK)
    col4 = colp.reshape(NW, 8, EPW_P // 8)
    ew4 = ewp.reshape(NW, 8, EPW_P // 8)
    x_pad = jnp.pad(x, ((0, N_PAD - N_NODES), (0, 0)))

    h = _h_kernel(x_pad, W0, b0.reshape(1, C_FEAT))
    degp = _deg_kernel(col4, ew4)
    dis = _dis_kernel(degp).reshape(N_PAD)

    p1 = _spmm_kernel(h, dis, row3, col3, ew3)
    out1 = _upd_kernel(p1, h, h, Wc0)
    p2 = _spmm_kernel(out1, dis, row3, col3, ew3)
    out2 = _upd_kernel(p2, h, out1, Wc1)
    return out2[:N_NODES]


# speedup vs baseline: 8.8163x; 1.0374x over previous
import functools

import jax
import jax.numpy as jnp
from jax import lax
from jax.experimental import pallas as pl
from jax.experimental.pallas import tpu as pltpu
from jax.experimental.pallas import tpu_sc as plsc

N_NODES = 10000
N_PAD = 10240
C_FEAT = 128
N_EDGES = 320000
ALPHA_C = 0.9

_INFO = plsc.get_sparse_core_info()
NC = _INFO.num_cores
NS = _INFO.num_subcores
NW = NC * NS
EPW = N_EDGES // NW
EPW_P = 10240
CHUNK = 64
NCHUNK = EPW_P // CHUNK
GRP = 8
NGRP = NCHUNK // GRP
NBUF = 4
RPT = N_PAD // NS
BLK = 1024


@functools.partial(
    pl.kernel,
    out_type=jax.ShapeDtypeStruct((NW, 8, N_PAD), jnp.float32),
    mesh=plsc.VectorSubcoreMesh(core_axis_name="c", subcore_axis_name="s"),
    compiler_params=pltpu.CompilerParams(needs_layout_passes=False),
    scratch_types=[
        pltpu.VMEM((8, EPW_P // 8), jnp.int32),
        pltpu.VMEM((8, EPW_P // 8), jnp.float32),
        pltpu.VMEM((1, N_PAD), jnp.float32),
    ],
)
def _deg_kernel(col_hbm, ew_hbm, out_hbm, col_v, ew_v, deg_v):
    cid = lax.axis_index("c")
    sid = lax.axis_index("s")
    wid = cid * NS + sid

    zeros16f = jnp.zeros((16,), jnp.float32)
    zeros16i = jnp.zeros((16,), jnp.int32)

    def zero_body(i, carry):
        deg_v[0, pl.ds(pl.multiple_of(i * 16, 16), 16)] = zeros16f
        return carry

    lax.fori_loop(0, N_PAD // 16, zero_body, 0)

    pltpu.sync_copy(col_hbm.at[wid], col_v)
    pltpu.sync_copy(ew_hbm.at[wid], ew_v)

    for r in range(8):
        def acc_body(i, carry):
            off = pl.multiple_of(i * 16, 16)
            idx = col_v[r, pl.ds(off, 16)]
            w = ew_v[r, pl.ds(off, 16)]
            plsc.addupdate_scatter(deg_v, [zeros16i, idx], w)
            return carry

        lax.fori_loop(0, EPW_P // 8 // 16, acc_body, 0)

    pltpu.sync_copy(deg_v, out_hbm.at[wid, pl.ds(0, 1)])


@functools.partial(
    pl.kernel,
    out_type=jax.ShapeDtypeStruct((NC, N_PAD, C_FEAT), jnp.float32),
    mesh=plsc.VectorSubcoreMesh(core_axis_name="c", subcore_axis_name="s"),
    compiler_params=pltpu.CompilerParams(needs_layout_passes=False),
    scratch_types=[
        pltpu.VMEM((2, GRP, CHUNK), jnp.int32),
        pltpu.VMEM((2, GRP, CHUNK), jnp.int32),
        pltpu.VMEM((2, GRP, CHUNK), jnp.float32),
        pltpu.VMEM((N_NODES,), jnp.float32),
        pltpu.VMEM((CHUNK,), jnp.float32),
        pltpu.VMEM((CHUNK, C_FEAT), jnp.float32),
        pltpu.VMEM((CHUNK, C_FEAT), jnp.float32),
        pltpu.VMEM((CHUNK, C_FEAT), jnp.float32),
        pltpu.VMEM((CHUNK, C_FEAT), jnp.float32),
        pltpu.VMEM_SHARED((N_PAD, C_FEAT), jnp.float32),
        pltpu.SemaphoreType.DMA,
        pltpu.SemaphoreType.DMA,
        pltpu.SemaphoreType.DMA,
        pltpu.SemaphoreType.DMA,
        pltpu.SemaphoreType.DMA,
        pltpu.SemaphoreType.DMA,
        pltpu.SemaphoreType.DMA,
        pltpu.SemaphoreType.DMA,
        pltpu.SemaphoreType.DMA,
    ],
)
def _spmm_kernel(h_hbm, dis_hbm, row_hbm, col_hbm, ew_hbm, out_hbm,
                 rv3, cv3, wv3, dis_v, nrm_v, gb0, gb1, gb2, gb3, accum,
                 gsem0, gsem1, gsem2, gsem3,
                 ssem0, ssem1, ssem2, ssem3, stsem):
    cid = lax.axis_index("c")
    sid = lax.axis_index("s")
    wid = cid * NS + sid
    gb = (gb0, gb1, gb2, gb3)
    gsem = (gsem0, gsem1, gsem2, gsem3)
    ssem = (ssem0, ssem1, ssem2, ssem3)

    pltpu.sync_copy(dis_hbm.at[pl.ds(0, N_NODES)], dis_v)

    zeros16 = jnp.zeros((16,), jnp.float32)

    def zzero(i, carry):
        for q in range(C_FEAT // 16):
            gb0[i, pl.ds(q * 16, 16)] = zeros16
        return carry

    lax.fori_loop(0, CHUNK, zzero, 0)
    for k in range(RPT // CHUNK):
        r0 = pl.multiple_of(sid * RPT + k * CHUNK, 8)
        pltpu.sync_copy(gb0, accum.at[pl.ds(r0, CHUNK)])
    plsc.subcore_barrier()

    def stage_grp(g1, dst, sync):
        o = pl.multiple_of(g1 * GRP, 8)
        copies = ((row_hbm, rv3), (col_hbm, cv3), (ew_hbm, wv3))
        for src, d3 in copies:
            c = pltpu.make_async_copy(src.at[wid, pl.ds(o, GRP)],
                                      d3.at[dst], stsem)
            if sync == "start":
                c.start()
            else:
                c.wait()

    stage_grp(0, 0, "start")
    stage_grp(0, 0, "wait")
    for k in range(NBUF - 1):
        pltpu.async_copy(h_hbm.at[rv3.at[0, k]], gb[k], gsem[k])

    def grp_body(g, carry):
        p = g % 2
        np_ = 1 - p

        for jj in range(GRP):
            b = jj % NBUF
            pltpu.make_async_copy(
                h_hbm.at[rv3.at[p, jj]], gb[b], gsem[b]).wait()

            if jj == 1:
                pl.when(g < NGRP - 1)(
                    lambda: stage_grp(g + 1, np_, "start"))

            tb = (jj + NBUF - 1) % NBUF

            def drain_tb():
                pltpu.make_async_copy(
                    gb[tb], accum.at[cv3.at[p, jj]], ssem[tb]).wait()

            if jj <= GRP - NBUF:
                if jj == 0:
                    pl.when(g > 0)(drain_tb)
                else:
                    drain_tb()
                pltpu.async_copy(h_hbm.at[rv3.at[p, jj + NBUF - 1]],
                                 gb[tb], gsem[tb])
            else:
                def next_group_gather():
                    if jj == GRP - NBUF + 1:
                        stage_grp(g + 1, np_, "wait")
                    drain_tb()
                    pltpu.async_copy(
                        h_hbm.at[rv3.at[np_, jj + NBUF - 1 - GRP]],
                        gb[tb], gsem[tb])
                pl.when(g < NGRP - 1)(next_group_gather)

            for i in range(CHUNK // 16):
                idr = rv3[p, jj, pl.ds(i * 16, 16)]
                idc = cv3[p, jj, pl.ds(i * 16, 16)]
                w16 = wv3[p, jj, pl.ds(i * 16, 16)]
                dr = plsc.load_gather(dis_v, [idr])
                dc = plsc.load_gather(dis_v, [idc])
                nrm_v[pl.ds(i * 16, 16)] = dr * w16 * dc

            gbb = gb[b]

            def scale_body(i, c2):
                off = pl.multiple_of(i * 16, 16)
                nv16 = nrm_v[pl.ds(off, 16)]
                for l in range(16):
                    nv = jnp.full((16,), nv16[l], jnp.float32)
                    e = off + l
                    for q in range(C_FEAT // 16):
                        gbb[e, pl.ds(q * 16, 16)] = (
                            gbb[e, pl.ds(q * 16, 16)] * nv)
                return c2

            lax.fori_loop(0, CHUNK // 16, scale_body, 0)

            pltpu.async_copy(gb[b], accum.at[cv3.at[p, jj]],
                             ssem[b], add=True)
        return carry

    lax.fori_loop(0, NGRP, grp_body, 0)

    for k in range(NBUF):
        pltpu.make_async_copy(gb[k], accum.at[cv3.at[0, 0]], ssem[k]).wait()
    plsc.subcore_barrier()

    def wout(k, carry):
        r0 = pl.multiple_of(sid * RPT + k * CHUNK, 8)
        pltpu.sync_copy(accum.at[pl.ds(r0, CHUNK)],
                        out_hbm.at[cid, pl.ds(r0, CHUNK)])
        return carry

    lax.fori_loop(0, RPT // CHUNK, wout, 0)


def _h_body(x_ref, w_ref, b_ref, o_ref):
    o_ref[...] = jnp.maximum(
        jnp.dot(x_ref[...], w_ref[...], preferred_element_type=jnp.float32)
        + b_ref[...], 0.0)


def _h_kernel(x, W0, b0):
    return pl.pallas_call(
        _h_body,
        grid=(N_PAD // BLK,),
        in_specs=[
            pl.BlockSpec((BLK, C_FEAT), lambda i: (i, 0)),
            pl.BlockSpec((C_FEAT, C_FEAT), lambda i: (0, 0)),
            pl.BlockSpec((1, C_FEAT), lambda i: (0, 0)),
        ],
        out_specs=pl.BlockSpec((BLK, C_FEAT), lambda i: (i, 0)),
        out_shape=jax.ShapeDtypeStruct((N_PAD, C_FEAT), jnp.float32),
    )(x, W0, b0)


def _dis_body(degp_ref, o_ref):
    deg = jnp.sum(degp_ref[:, 0, :], axis=0, keepdims=True)
    safe = jnp.where(deg > 0, deg, 1.0)
    o_ref[...] = jnp.where(deg > 0, lax.rsqrt(safe), 0.0)


def _dis_kernel(degp):
    return pl.pallas_call(
        _dis_body,
        grid=(1,),
        in_specs=[pl.BlockSpec((NW, 8, N_PAD), lambda i: (0, 0, 0))],
        out_specs=pl.BlockSpec((1, N_PAD), lambda i: (0, 0)),
        out_shape=jax.ShapeDtypeStruct((1, N_PAD), jnp.float32),
    )(degp)


def _upd_body(p_ref, h_ref, prev_ref, w_ref, o_ref):
    agg = p_ref[0, :, :] + p_ref[1, :, :]
    t = (1.0 - ALPHA_C) * agg + ALPHA_C * h_ref[...]
    o_ref[...] = prev_ref[...] + jnp.maximum(
        jnp.dot(t, w_ref[...], preferred_element_type=jnp.float32), 0.0)


def _upd_kernel(p, h, prev, W):
    return pl.pallas_call(
        _upd_body,
        grid=(N_PAD // BLK,),
        in_specs=[
            pl.BlockSpec((NC, BLK, C_FEAT), lambda i: (0, i, 0)),
            pl.BlockSpec((BLK, C_FEAT), lambda i: (i, 0)),
            pl.BlockSpec((BLK, C_FEAT), lambda i: (i, 0)),
            pl.BlockSpec((C_FEAT, C_FEAT), lambda i: (0, 0)),
        ],
        out_specs=pl.BlockSpec((BLK, C_FEAT), lambda i: (i, 0)),
        out_shape=jax.ShapeDtypeStruct((N_PAD, C_FEAT), jnp.float32),
    )(p, h, prev, W)


def kernel(x, edge_index, edge_weight, edge_attr, W0, b0, Wc0, Wc1):
    row2 = edge_index[0].reshape(NW, EPW)
    col2 = edge_index[1].reshape(NW, EPW)
    ew2 = edge_weight.reshape(NW, EPW)
    pad = ((0, 0), (0, EPW_P - EPW))
    rowp = jnp.pad(row2, pad)
    colp = jnp.pad(col2, pad)
    ewp = jnp.pad(ew2, pad)

    row3 = rowp.reshape(NW, NCHUNK, CHUNK)
    col3 = colp.reshape(NW, NCHUNK, CHUNK)
    ew3 = ewp.reshape(NW, NCHUNK, CHUNK)
    col4 = colp.reshape(NW, 8, EPW_P // 8)
    ew4 = ewp.reshape(NW, 8, EPW_P // 8)
    x_pad = jnp.pad(x, ((0, N_PAD - N_NODES), (0, 0)))

    h = _h_kernel(x_pad, W0, b0.reshape(1, C_FEAT))
    degp = _deg_kernel(col4, ew4)
    dis = _dis_kernel(degp).reshape(N_PAD)

    p1 = _spmm_kernel(h, dis, row3, col3, ew3)
    out1 = _upd_kernel(p1, h, h, Wc0)
    p2 = _spmm_kernel(out1, dis, row3, col3, ew3)
    out2 = _upd_kernel(p2, h, out1, Wc1)
    return out2[:N_NODES]


# speedup vs baseline: 8.8394x; 1.0026x over previous
import functools

import jax
import jax.numpy as jnp
from jax import lax
from jax.experimental import pallas as pl
from jax.experimental.pallas import tpu as pltpu
from jax.experimental.pallas import tpu_sc as plsc

N_NODES = 10000
N_PAD = 10240
C_FEAT = 128
N_EDGES = 320000
ALPHA_C = 0.9

_INFO = plsc.get_sparse_core_info()
NC = _INFO.num_cores
NS = _INFO.num_subcores
NW = NC * NS
EPW = N_EDGES // NW
EPW_P = 10240
CHUNK = 64
NCHUNK = EPW_P // CHUNK
GRP = 8
NGRP = NCHUNK // GRP
NBUF = 4
RPT = N_PAD // NS
BLK = 1024


@functools.partial(
    pl.kernel,
    out_type=jax.ShapeDtypeStruct((NW, 8, N_PAD), jnp.float32),
    mesh=plsc.VectorSubcoreMesh(core_axis_name="c", subcore_axis_name="s"),
    compiler_params=pltpu.CompilerParams(needs_layout_passes=False),
    scratch_types=[
        pltpu.VMEM((8, EPW_P // 8), jnp.int32),
        pltpu.VMEM((8, EPW_P // 8), jnp.float32),
        pltpu.VMEM((1, N_PAD), jnp.float32),
    ],
)
def _deg_kernel(col_hbm, ew_hbm, out_hbm, col_v, ew_v, deg_v):
    cid = lax.axis_index("c")
    sid = lax.axis_index("s")
    wid = cid * NS + sid

    zeros16f = jnp.zeros((16,), jnp.float32)
    zeros16i = jnp.zeros((16,), jnp.int32)

    def zero_body(i, carry):
        deg_v[0, pl.ds(pl.multiple_of(i * 16, 16), 16)] = zeros16f
        return carry

    lax.fori_loop(0, N_PAD // 16, zero_body, 0)

    pltpu.sync_copy(col_hbm.at[wid], col_v)
    pltpu.sync_copy(ew_hbm.at[wid], ew_v)

    for r in range(8):
        def acc_body(i, carry):
            off = pl.multiple_of(i * 16, 16)
            idx = col_v[r, pl.ds(off, 16)]
            w = ew_v[r, pl.ds(off, 16)]
            plsc.addupdate_scatter(deg_v, [zeros16i, idx], w)
            return carry

        lax.fori_loop(0, EPW_P // 8 // 16, acc_body, 0)

    pltpu.sync_copy(deg_v, out_hbm.at[wid, pl.ds(0, 1)])


@functools.partial(
    pl.kernel,
    out_type=jax.ShapeDtypeStruct((NC, N_PAD, C_FEAT), jnp.float32),
    mesh=plsc.VectorSubcoreMesh(core_axis_name="c", subcore_axis_name="s"),
    compiler_params=pltpu.CompilerParams(needs_layout_passes=False, use_tc_tiling_on_sc=False),
    scratch_types=[
        pltpu.VMEM((2, GRP, CHUNK), jnp.int32),
        pltpu.VMEM((2, GRP, CHUNK), jnp.int32),
        pltpu.VMEM((2, GRP, CHUNK), jnp.float32),
        pltpu.VMEM((N_NODES,), jnp.float32),
        pltpu.VMEM((CHUNK,), jnp.float32),
        pltpu.VMEM((CHUNK, C_FEAT), jnp.float32),
        pltpu.VMEM((CHUNK, C_FEAT), jnp.float32),
        pltpu.VMEM((CHUNK, C_FEAT), jnp.float32),
        pltpu.VMEM((CHUNK, C_FEAT), jnp.float32),
        pltpu.VMEM_SHARED((N_PAD, C_FEAT), jnp.float32),
        pltpu.SemaphoreType.DMA,
        pltpu.SemaphoreType.DMA,
        pltpu.SemaphoreType.DMA,
        pltpu.SemaphoreType.DMA,
        pltpu.SemaphoreType.DMA,
        pltpu.SemaphoreType.DMA,
        pltpu.SemaphoreType.DMA,
        pltpu.SemaphoreType.DMA,
        pltpu.SemaphoreType.DMA,
    ],
)
def _spmm_kernel(h_hbm, dis_hbm, row_hbm, col_hbm, ew_hbm, out_hbm,
                 rv3, cv3, wv3, dis_v, nrm_v, gb0, gb1, gb2, gb3, accum,
                 gsem0, gsem1, gsem2, gsem3,
                 ssem0, ssem1, ssem2, ssem3, stsem):
    cid = lax.axis_index("c")
    sid = lax.axis_index("s")
    wid = cid * NS + sid
    gb = (gb0, gb1, gb2, gb3)
    gsem = (gsem0, gsem1, gsem2, gsem3)
    ssem = (ssem0, ssem1, ssem2, ssem3)

    pltpu.sync_copy(dis_hbm.at[pl.ds(0, N_NODES)], dis_v)

    zeros16 = jnp.zeros((16,), jnp.float32)

    def zzero(i, carry):
        for q in range(C_FEAT // 16):
            gb0[i, pl.ds(q * 16, 16)] = zeros16
        return carry

    lax.fori_loop(0, CHUNK, zzero, 0)
    for k in range(RPT // CHUNK):
        r0 = pl.multiple_of(sid * RPT + k * CHUNK, 8)
        pltpu.sync_copy(gb0, accum.at[pl.ds(r0, CHUNK)])
    plsc.subcore_barrier()

    def stage_grp(g1, dst, sync):
        o = pl.multiple_of(g1 * GRP, 8)
        copies = ((row_hbm, rv3), (col_hbm, cv3), (ew_hbm, wv3))
        for src, d3 in copies:
            c = pltpu.make_async_copy(src.at[wid, pl.ds(o, GRP)],
                                      d3.at[dst], stsem)
            if sync == "start":
                c.start()
            else:
                c.wait()

    stage_grp(0, 0, "start")
    stage_grp(0, 0, "wait")
    for k in range(NBUF - 1):
        pltpu.async_copy(h_hbm.at[rv3.at[0, k]], gb[k], gsem[k])

    def grp_body(g, carry):
        p = g % 2
        np_ = 1 - p

        for jj in range(GRP):
            b = jj % NBUF
            pltpu.make_async_copy(
                h_hbm.at[rv3.at[p, jj]], gb[b], gsem[b]).wait()

            if jj == 1:
                pl.when(g < NGRP - 1)(
                    lambda: stage_grp(g + 1, np_, "start"))

            tb = (jj + NBUF - 1) % NBUF

            def drain_tb():
                pltpu.make_async_copy(
                    gb[tb], accum.at[cv3.at[p, jj]], ssem[tb]).wait()

            if jj <= GRP - NBUF:
                if jj == 0:
                    pl.when(g > 0)(drain_tb)
                else:
                    drain_tb()
                pltpu.async_copy(h_hbm.at[rv3.at[p, jj + NBUF - 1]],
                                 gb[tb], gsem[tb])
            else:
                def next_group_gather():
                    if jj == GRP - NBUF + 1:
                        stage_grp(g + 1, np_, "wait")
                    drain_tb()
                    pltpu.async_copy(
                        h_hbm.at[rv3.at[np_, jj + NBUF - 1 - GRP]],
                        gb[tb], gsem[tb])
                pl.when(g < NGRP - 1)(next_group_gather)

            for i in range(CHUNK // 16):
                idr = rv3[p, jj, pl.ds(i * 16, 16)]
                idc = cv3[p, jj, pl.ds(i * 16, 16)]
                w16 = wv3[p, jj, pl.ds(i * 16, 16)]
                dr = plsc.load_gather(dis_v, [idr])
                dc = plsc.load_gather(dis_v, [idc])
                nrm_v[pl.ds(i * 16, 16)] = dr * w16 * dc

            gbb = gb[b]

            def scale_body(i, c2):
                off = pl.multiple_of(i * 16, 16)
                nv16 = nrm_v[pl.ds(off, 16)]
                for l in range(16):
                    nv = jnp.full((16,), nv16[l], jnp.float32)
                    e = off + l
                    for q in range(C_FEAT // 16):
                        gbb[e, pl.ds(q * 16, 16)] = (
                            gbb[e, pl.ds(q * 16, 16)] * nv)
                return c2

            lax.fori_loop(0, CHUNK // 16, scale_body, 0)

            pltpu.async_copy(gb[b], accum.at[cv3.at[p, jj]],
                             ssem[b], add=True)
        return carry

    lax.fori_loop(0, NGRP, grp_body, 0)

    for k in range(NBUF):
        pltpu.make_async_copy(gb[k], accum.at[cv3.at[0, 0]], ssem[k]).wait()
    plsc.subcore_barrier()

    def wout(k, carry):
        r0 = pl.multiple_of(sid * RPT + k * CHUNK, 8)
        pltpu.sync_copy(accum.at[pl.ds(r0, CHUNK)],
                        out_hbm.at[cid, pl.ds(r0, CHUNK)])
        return carry

    lax.fori_loop(0, RPT // CHUNK, wout, 0)


def _h_body(x_ref, w_ref, b_ref, o_ref):
    o_ref[...] = jnp.maximum(
        jnp.dot(x_ref[...], w_ref[...], preferred_element_type=jnp.float32)
        + b_ref[...], 0.0)


def _h_kernel(x, W0, b0):
    return pl.pallas_call(
        _h_body,
        grid=(N_PAD // BLK,),
        in_specs=[
            pl.BlockSpec((BLK, C_FEAT), lambda i: (i, 0)),
            pl.BlockSpec((C_FEAT, C_FEAT), lambda i: (0, 0)),
            pl.BlockSpec((1, C_FEAT), lambda i: (0, 0)),
        ],
        out_specs=pl.BlockSpec((BLK, C_FEAT), lambda i: (i, 0)),
        out_shape=jax.ShapeDtypeStruct((N_PAD, C_FEAT), jnp.float32),
    )(x, W0, b0)


def _dis_body(degp_ref, o_ref):
    deg = jnp.sum(degp_ref[:, 0, :], axis=0, keepdims=True)
    safe = jnp.where(deg > 0, deg, 1.0)
    o_ref[...] = jnp.where(deg > 0, lax.rsqrt(safe), 0.0)


def _dis_kernel(degp):
    return pl.pallas_call(
        _dis_body,
        grid=(1,),
        in_specs=[pl.BlockSpec((NW, 8, N_PAD), lambda i: (0, 0, 0))],
        out_specs=pl.BlockSpec((1, N_PAD), lambda i: (0, 0)),
        out_shape=jax.ShapeDtypeStruct((1, N_PAD), jnp.float32),
    )(degp)


def _upd_body(p_ref, h_ref, prev_ref, w_ref, o_ref):
    agg = p_ref[0, :, :] + p_ref[1, :, :]
    t = (1.0 - ALPHA_C) * agg + ALPHA_C * h_ref[...]
    o_ref[...] = prev_ref[...] + jnp.maximum(
        jnp.dot(t, w_ref[...], preferred_element_type=jnp.float32), 0.0)


def _upd_kernel(p, h, prev, W):
    return pl.pallas_call(
        _upd_body,
        grid=(N_PAD // BLK,),
        in_specs=[
            pl.BlockSpec((NC, BLK, C_FEAT), lambda i: (0, i, 0)),
            pl.BlockSpec((BLK, C_FEAT), lambda i: (i, 0)),
            pl.BlockSpec((BLK, C_FEAT), lambda i: (i, 0)),
            pl.BlockSpec((C_FEAT, C_FEAT), lambda i: (0, 0)),
        ],
        out_specs=pl.BlockSpec((BLK, C_FEAT), lambda i: (i, 0)),
        out_shape=jax.ShapeDtypeStruct((N_PAD, C_FEAT), jnp.float32),
    )(p, h, prev, W)


def kernel(x, edge_index, edge_weight, edge_attr, W0, b0, Wc0, Wc1):
    row2 = edge_index[0].reshape(NW, EPW)
    col2 = edge_index[1].reshape(NW, EPW)
    ew2 = edge_weight.reshape(NW, EPW)
    pad = ((0, 0), (0, EPW_P - EPW))
    rowp = jnp.pad(row2, pad)
    colp = jnp.pad(col2, pad)
    ewp = jnp.pad(ew2, pad)

    row3 = rowp.reshape(NW, NCHUNK, CHUNK)
    col3 = colp.reshape(NW, NCHUNK, CHUNK)
    ew3 = ewp.reshape(NW, NCHUNK, CHUNK)
    col4 = colp.reshape(NW, 8, EPW_P // 8)
    ew4 = ewp.reshape(NW, 8, EPW_P // 8)
    x_pad = jnp.pad(x, ((0, N_PAD - N_NODES), (0, 0)))

    h = _h_kernel(x_pad, W0, b0.reshape(1, C_FEAT))
    degp = _deg_kernel(col4, ew4)
    dis = _dis_kernel(degp).reshape(N_PAD)

    p1 = _spmm_kernel(h, dis, row3, col3, ew3)
    out1 = _upd_kernel(p1, h, h, Wc0)
    p2 = _spmm_kernel(out1, dis, row3, col3, ew3)
    out2 = _upd_kernel(p2, h, out1, Wc1)
    return out2[:N_NODES]
